# Initial kernel scaffold; baseline (speedup 1.0000x reference)
import jax
import jax.numpy as jnp
from jax.experimental import pallas as pl


def kernel(input_seq, last_hidden, nodes, batch, enc_W, enc_b, Wih, Whh, bih, bhh, proj_W, proj_b, comp_W, comp_b, out_W, out_b):
    raise NotImplementedError("write your pallas kernel here")


# speedup vs baseline: 70.3157x; 70.3157x over previous
import jax
import jax.numpy as jnp
from jax import lax
from jax.experimental import pallas as pl


def _mm_t(a, b):
    return lax.dot_general(a, b, (((1,), (1,)), ((), ())),
                           preferred_element_type=jnp.float32)


def _fused_body(iseq_ref, h_ref, nodes_ref, batch_ref, encW_ref, encb_ref,
                Wih_ref, Whh_ref, bih_ref, bhh_ref, projW_ref, projb_ref,
                compW_ref, compb_ref, outW_ref, outb_ref,
                out_ref, hid_ref, attn_ref):
    H = h_ref.shape[1]
    bs = h_ref.shape[0]

    x = jnp.maximum(_mm_t(iseq_ref[0], encW_ref[...]) + encb_ref[...], 0.0)

    h = h_ref[...]
    gx = _mm_t(x, Wih_ref[...]) + bih_ref[...]
    gh = _mm_t(h, Whh_ref[...]) + bhh_ref[...]
    r = jax.nn.sigmoid(gx[:, :H] + gh[:, :H])
    z = jax.nn.sigmoid(gx[:, H:2 * H] + gh[:, H:2 * H])
    n = jnp.tanh(gx[:, 2 * H:] + r * gh[:, 2 * H:])
    h_new = (1.0 - z) * n + z * h
    hid_ref[...] = h_new

    rnn_out = _mm_t(jnp.maximum(h_new, 0.0), projW_ref[...]) + projb_ref[...]

    logits = _mm_t(rnn_out, nodes_ref[...])

    batch_row = batch_ref[...]

    NEG = jnp.float32(-1e30)
    M = jnp.zeros_like(logits)
    masks = []
    for s in range(bs):
        mask = batch_row == s
        masks.append(mask)
        m_s = jnp.max(jnp.where(mask, logits, NEG), axis=1, keepdims=True)
        M = M + jnp.where(mask, m_s, 0.0)

    ex = jnp.exp(logits - M)

    S = jnp.zeros_like(logits)
    for s in range(bs):
        mask = masks[s]
        s_s = jnp.sum(jnp.where(mask, ex, 0.0), axis=1, keepdims=True)
        S = S + jnp.where(mask, s_s, 0.0)

    attn_w = ex / S
    attn_ref[...] = attn_w

    rowid = lax.broadcasted_iota(jnp.int32, logits.shape, 0)
    wdiag = jnp.where(rowid == batch_row, attn_w, 0.0)
    context = jnp.dot(wdiag, nodes_ref[...], preferred_element_type=jnp.float32)

    concat = jnp.concatenate([rnn_out, context, x], axis=1)
    co = jnp.maximum(_mm_t(concat, compW_ref[...]) + compb_ref[...], 0.0)
    out_ref[...] = _mm_t(co, outW_ref[...]) + outb_ref[...]


def kernel(input_seq, last_hidden, nodes, batch, enc_W, enc_b, Wih, Whh,
           bih, bhh, proj_W, proj_b, comp_W, comp_b, out_W, out_b):
    n_nodes, H = nodes.shape
    bs = input_seq.shape[1]
    out_dim = out_W.shape[0]

    batch2 = batch.reshape(1, n_nodes)
    row = lambda v: v.reshape(1, -1)

    out, h_new, attn_w = pl.pallas_call(
        _fused_body,
        out_shape=[
            jax.ShapeDtypeStruct((bs, out_dim), jnp.float32),
            jax.ShapeDtypeStruct((bs, H), jnp.float32),
            jax.ShapeDtypeStruct((bs, n_nodes), jnp.float32),
        ],
    )(input_seq, last_hidden[0], nodes, batch2, enc_W, row(enc_b),
      Wih, Whh, row(bih), row(bhh), proj_W, row(proj_b),
      comp_W, row(comp_b), out_W, row(out_b))

    return out, h_new[None], attn_w
